# Initial kernel scaffold; baseline (speedup 1.0000x reference)
import jax
import jax.numpy as jnp
from jax.experimental import pallas as pl


def kernel(x, edge_index, ggc_weight, gru_w_ih, gru_w_hh, gru_b_ih, gru_b_hh, lin_w, lin_b):
    raise NotImplementedError("write your pallas kernel here")


# speedup vs baseline: 6.7058x; 6.7058x over previous
import jax
import jax.numpy as jnp
from jax import lax
from jax.experimental import pallas as pl
from jax.experimental.pallas import tpu as pltpu
from jax.experimental.pallas import tpu_sc as plsc

N_NODES = 10000
N_EDGES = 320000
D = 128

_NC = 2
_NS = 16
_NW = _NC * _NS
_EB = 80
_EPW = N_EDGES // _NW
_NB = _EPW // _EB
_RPT = N_NODES // _NS
_ZC = 125
_NZ = _RPT // _ZC

_BLK = 400
_NBLK = N_NODES // _BLK


def _mp_body(m_hbm, srcb_hbm, dstb_hbm, out_hbm,
             agg_sh, zb, si0, di0, rows0, si1, di1, rows1, sem0, sem1):
    c = lax.axis_index("c")
    s = lax.axis_index("s")
    wid = s * _NC + c
    row0 = wid * _NB

    zv = jnp.zeros((16,), jnp.float32)

    def _zrow(r, carry):
        for c16 in range(D // 16):
            zb[r, pl.ds(c16 * 16, 16)] = zv
        return carry

    lax.fori_loop(0, _ZC, _zrow, 0)
    base_row = s * _RPT
    for k in range(_NZ):
        pltpu.sync_copy(zb, agg_sh.at[pl.ds(base_row + k * _ZC, _ZC)])
    plsc.subcore_barrier()

    bufs = ((si0, di0, rows0, sem0), (si1, di1, rows1, sem1))

    def _start(g, b):
        si, di, rows, sem = bufs[b]
        pltpu.sync_copy(srcb_hbm.at[row0 + g], si)
        pltpu.sync_copy(dstb_hbm.at[row0 + g], di)
        pltpu.async_copy(m_hbm.at[si], rows, sem)

    def _finish(b):
        si, di, rows, sem = bufs[b]
        pltpu.make_async_copy(m_hbm.at[si], rows, sem).wait()
        pltpu.sync_copy(rows, agg_sh.at[di], add=True)

    _start(0, 0)
    _start(1, 1)

    def _body(i, carry):
        g = 2 * i
        _finish(0)
        _start(g + 2, 0)
        _finish(1)
        _start(g + 3, 1)
        return carry

    lax.fori_loop(0, (_NB - 3) // 2, _body, 0)
    _finish(0)
    _start(_NB - 1, 0)
    _finish(1)
    _finish(0)

    plsc.subcore_barrier()
    w0 = pl.multiple_of(s * 624, 8)

    @pl.when(s < _NS - 1)
    def _():
        pltpu.sync_copy(agg_sh.at[pl.ds(w0, 624)],
                        out_hbm.at[c, pl.ds(w0, 624)])

    @pl.when(s == _NS - 1)
    def _():
        pltpu.sync_copy(agg_sh.at[pl.ds((_NS - 1) * 624, N_NODES - (_NS - 1) * 624)],
                        out_hbm.at[c, pl.ds((_NS - 1) * 624, N_NODES - (_NS - 1) * 624)])


_mp = pl.kernel(
    _mp_body,
    out_type=jax.ShapeDtypeStruct((_NC, N_NODES, D), jnp.float32),
    mesh=plsc.VectorSubcoreMesh(core_axis_name="c", subcore_axis_name="s"),
    scratch_types=[
        pltpu.VMEM_SHARED((N_NODES, D), jnp.float32),
        pltpu.VMEM((_ZC, D), jnp.float32),
        pltpu.VMEM((_EB,), jnp.int32),
        pltpu.VMEM((_EB,), jnp.int32),
        pltpu.VMEM((_EB, D), jnp.float32),
        pltpu.VMEM((_EB,), jnp.int32),
        pltpu.VMEM((_EB,), jnp.int32),
        pltpu.VMEM((_EB, D), jnp.float32),
        pltpu.SemaphoreType.DMA,
        pltpu.SemaphoreType.DMA,
    ],
)


def _mm_body(h_ref, w_ref, o_ref):
    o_ref[...] = jnp.dot(h_ref[...], w_ref[...],
                         preferred_element_type=jnp.float32)


def _mm(h, w):
    return pl.pallas_call(
        _mm_body,
        grid=(_NBLK,),
        in_specs=[pl.BlockSpec((_BLK, D), lambda i: (i, 0)),
                  pl.BlockSpec((D, D), lambda i: (0, 0))],
        out_specs=pl.BlockSpec((_BLK, D), lambda i: (i, 0)),
        out_shape=jax.ShapeDtypeStruct((N_NODES, D), jnp.float32),
    )(h, w)


def _gru_block(parts_ref, h_ref, wih_ref, whh_ref, bih_ref, bhh_ref):
    agg = parts_ref[0] + parts_ref[1]
    h = h_ref[...]
    gi = jnp.dot(agg, wih_ref[...], preferred_element_type=jnp.float32) + bih_ref[...]
    gh = jnp.dot(h, whh_ref[...], preferred_element_type=jnp.float32) + bhh_ref[...]
    r = jax.nn.sigmoid(gi[:, :D] + gh[:, :D])
    z = jax.nn.sigmoid(gi[:, D:2 * D] + gh[:, D:2 * D])
    n = jnp.tanh(gi[:, 2 * D:] + r * gh[:, 2 * D:])
    return (1.0 - z) * n + z * h


def _gru_mm_body(parts_ref, h_ref, wih_ref, whh_ref, bih_ref, bhh_ref,
                 wnext_ref, h_out, m_out):
    hn = _gru_block(parts_ref, h_ref, wih_ref, whh_ref, bih_ref, bhh_ref)
    h_out[...] = hn
    m_out[...] = jnp.dot(hn, wnext_ref[...], preferred_element_type=jnp.float32)


def _gru_mm(parts, h, wihT, whhT, bih, bhh, wnext):
    return pl.pallas_call(
        _gru_mm_body,
        grid=(_NBLK,),
        in_specs=[pl.BlockSpec((_NC, _BLK, D), lambda i: (0, i, 0)),
                  pl.BlockSpec((_BLK, D), lambda i: (i, 0)),
                  pl.BlockSpec((D, 3 * D), lambda i: (0, 0)),
                  pl.BlockSpec((D, 3 * D), lambda i: (0, 0)),
                  pl.BlockSpec((1, 3 * D), lambda i: (0, 0)),
                  pl.BlockSpec((1, 3 * D), lambda i: (0, 0)),
                  pl.BlockSpec((D, D), lambda i: (0, 0))],
        out_specs=[pl.BlockSpec((_BLK, D), lambda i: (i, 0)),
                   pl.BlockSpec((_BLK, D), lambda i: (i, 0))],
        out_shape=[jax.ShapeDtypeStruct((N_NODES, D), jnp.float32),
                   jax.ShapeDtypeStruct((N_NODES, D), jnp.float32)],
    )(parts, h, wihT, whhT, bih, bhh, wnext)


def _gru_max_body(parts_ref, h_ref, wih_ref, whh_ref, bih_ref, bhh_ref, mx_out):
    hn = _gru_block(parts_ref, h_ref, wih_ref, whh_ref, bih_ref, bhh_ref)
    lr = jnp.where(hn >= 0.0, hn, 0.01 * hn)
    mx_out[...] = jnp.max(lr, axis=0, keepdims=True)[None]


def _gru_max(parts, h, wihT, whhT, bih, bhh):
    return pl.pallas_call(
        _gru_max_body,
        grid=(_NBLK,),
        in_specs=[pl.BlockSpec((_NC, _BLK, D), lambda i: (0, i, 0)),
                  pl.BlockSpec((_BLK, D), lambda i: (i, 0)),
                  pl.BlockSpec((D, 3 * D), lambda i: (0, 0)),
                  pl.BlockSpec((D, 3 * D), lambda i: (0, 0)),
                  pl.BlockSpec((1, 3 * D), lambda i: (0, 0)),
                  pl.BlockSpec((1, 3 * D), lambda i: (0, 0))],
        out_specs=pl.BlockSpec((1, 1, D), lambda i: (i, 0, 0)),
        out_shape=jax.ShapeDtypeStruct((_NBLK, 1, D), jnp.float32),
    )(parts, h, wihT, whhT, bih, bhh)


def _fin_body(mx_ref, lw_ref, lb_ref, o_ref):
    g = jnp.max(mx_ref[...], axis=0, keepdims=True)
    o_ref[...] = jax.nn.sigmoid(
        jnp.dot(g, lw_ref[...], preferred_element_type=jnp.float32) + lb_ref[...])


def _fin(mx, lwT, lb):
    return pl.pallas_call(
        _fin_body,
        out_shape=jax.ShapeDtypeStruct((1, D), jnp.float32),
    )(mx, lwT, lb)


def kernel(x, edge_index, ggc_weight, gru_w_ih, gru_w_hh, gru_b_ih, gru_b_hh,
           lin_w, lin_b):
    srcb = edge_index[0].reshape(_NW * _NB, _EB)
    dstb = edge_index[1].reshape(_NW * _NB, _EB)
    wihT = gru_w_ih.T
    whhT = gru_w_hh.T
    bih = gru_b_ih.reshape(1, 3 * D)
    bhh = gru_b_hh.reshape(1, 3 * D)

    h = x
    m = _mm(h, ggc_weight[0])
    parts = _mp(m, srcb, dstb)
    h, m = _gru_mm(parts, h, wihT, whhT, bih, bhh, ggc_weight[1])
    parts = _mp(m, srcb, dstb)
    mx = _gru_max(parts, h, wihT, whhT, bih, bhh).reshape(_NBLK, D)
    return _fin(mx, lin_w.T, lin_b.reshape(1, D))


# speedup vs baseline: 9.8531x; 1.4693x over previous
import jax
import jax.numpy as jnp
from jax import lax
from jax.experimental import pallas as pl
from jax.experimental.pallas import tpu as pltpu
from jax.experimental.pallas import tpu_sc as plsc

N_NODES = 10000
N_EDGES = 320000
D = 128

_NC = 2
_NS = 16
_NW = _NC * _NS
_EB = 80
_EPW = N_EDGES // _NW
_NB = _EPW // _EB
_RPT = N_NODES // _NS
_ZC = 25
_NZ = _RPT // _ZC

_BLK = 400
_NBLK = N_NODES // _BLK


def _mp_body(m_hbm, srcb_hbm, dstb_hbm, out_hbm,
             agg_sh, zb,
             si0, si1, si2, si3, si4, si5,
             di0, di1, di2, di3, di4, di5,
             rows0, rows1, rows2, rows3,
             isem0, isem1, isem2, isem3, isem4, isem5,
             gsem0, gsem1, gsem2, gsem3,
             ssem0, ssem1, ssem2, ssem3):
    c = lax.axis_index("c")
    s = lax.axis_index("s")
    wid = s * _NC + c
    row0 = wid * _NB

    si = (si0, si1, si2, si3, si4, si5)
    di = (di0, di1, di2, di3, di4, di5)
    isem = (isem0, isem1, isem2, isem3, isem4, isem5)
    rows = (rows0, rows1, rows2, rows3)
    gsem = (gsem0, gsem1, gsem2, gsem3)
    ssem = (ssem0, ssem1, ssem2, ssem3)

    def i_start(g, k):
        pltpu.async_copy(srcb_hbm.at[row0 + g], si[k], isem[k])
        pltpu.async_copy(dstb_hbm.at[row0 + g], di[k], isem[k])

    def i_wait(g, k):
        pltpu.make_async_copy(srcb_hbm.at[row0 + g], si[k], isem[k]).wait()
        pltpu.make_async_copy(dstb_hbm.at[row0 + g], di[k], isem[k]).wait()

    def g_start(g, b, k):
        pltpu.async_copy(m_hbm.at[si[k]], rows[b], gsem[b])

    def g_wait(g, b, k):
        pltpu.make_async_copy(m_hbm.at[si[k]], rows[b], gsem[b]).wait()

    def s_start(g, b, k):
        pltpu.async_copy(rows[b], agg_sh.at[di[k]], ssem[b], add=True)

    def s_wait(g, b, k):
        pltpu.make_async_copy(rows[b], agg_sh.at[di[k]], ssem[b]).wait()

    for g in range(4):
        i_start(g, g % 6)
    zv = jnp.zeros((16,), jnp.float32)

    def _zrow(r, carry):
        for c16 in range(D // 16):
            zb[r, pl.ds(c16 * 16, 16)] = zv
        return carry

    lax.fori_loop(0, _ZC, _zrow, 0)
    base_row = s * _RPT
    for k in range(_NZ):
        pltpu.sync_copy(zb, agg_sh.at[pl.ds(base_row + k * _ZC, _ZC)])
    plsc.subcore_barrier()

    i_wait(0, 0)
    g_start(0, 0, 0)
    i_wait(1, 1)
    g_start(1, 1, 1)
    for g in range(2):
        g_wait(g, g % 4, g % 6)
        s_start(g, g % 4, g % 6)
        i_wait(g + 2, (g + 2) % 6)
        g_start(g + 2, (g + 2) % 4, (g + 2) % 6)
        i_start(g + 4, (g + 4) % 6)

    niter = (_NB - 6) // 12
    def _body(it, carry):
        g0 = 2 + 12 * it
        for j in range(12):
            gg = g0 + j
            b = (2 + j) % 4
            k = (2 + j) % 6
            g_wait(gg, b, k)
            s_start(gg, b, k)
            s_wait(gg - 2, j % 4, j % 6)
            i_wait(gg + 2, (4 + j) % 6)
            g_start(gg + 2, j % 4, (4 + j) % 6)
            i_start(gg + 4, j % 6)
        return carry

    lax.fori_loop(0, niter, _body, 0)

    for gg in range(2 + 12 * niter, _NB):
        g_wait(gg, gg % 4, gg % 6)
        s_start(gg, gg % 4, gg % 6)
        s_wait(gg - 2, (gg - 2) % 4, (gg - 2) % 6)
        if gg + 2 < _NB:
            i_wait(gg + 2, (gg + 2) % 6)
            g_start(gg + 2, (gg + 2) % 4, (gg + 2) % 6)
        if gg + 4 < _NB:
            i_start(gg + 4, (gg + 4) % 6)
    for gg in range(_NB - 2, _NB):
        s_wait(gg, gg % 4, gg % 6)

    plsc.subcore_barrier()
    w0 = pl.multiple_of(s * 624, 8)

    @pl.when(s < _NS - 1)
    def _():
        pltpu.sync_copy(agg_sh.at[pl.ds(w0, 624)],
                        out_hbm.at[c, pl.ds(w0, 624)])

    @pl.when(s == _NS - 1)
    def _():
        pltpu.sync_copy(agg_sh.at[pl.ds((_NS - 1) * 624, N_NODES - (_NS - 1) * 624)],
                        out_hbm.at[c, pl.ds((_NS - 1) * 624, N_NODES - (_NS - 1) * 624)])


_mp = pl.kernel(
    _mp_body,
    out_type=jax.ShapeDtypeStruct((_NC, N_NODES, D), jnp.float32),
    mesh=plsc.VectorSubcoreMesh(core_axis_name="c", subcore_axis_name="s"),
    scratch_types=(
        [pltpu.VMEM_SHARED((N_NODES, D), jnp.float32)]
        + [pltpu.VMEM((_ZC, D), jnp.float32)]
        + [pltpu.VMEM((_EB,), jnp.int32)] * 12
        + [pltpu.VMEM((_EB, D), jnp.float32)] * 4
        + [pltpu.SemaphoreType.DMA] * 14
    ),
)


def _mm_body(h_ref, w_ref, o_ref):
    o_ref[...] = jnp.dot(h_ref[...], w_ref[...],
                         preferred_element_type=jnp.float32)


def _mm(h, w):
    return pl.pallas_call(
        _mm_body,
        grid=(_NBLK,),
        in_specs=[pl.BlockSpec((_BLK, D), lambda i: (i, 0)),
                  pl.BlockSpec((D, D), lambda i: (0, 0))],
        out_specs=pl.BlockSpec((_BLK, D), lambda i: (i, 0)),
        out_shape=jax.ShapeDtypeStruct((N_NODES, D), jnp.float32),
    )(h, w)


def _gru_block(parts_ref, h_ref, wih_ref, whh_ref, bih_ref, bhh_ref):
    agg = parts_ref[0] + parts_ref[1]
    h = h_ref[...]
    gi = jnp.dot(agg, wih_ref[...], preferred_element_type=jnp.float32) + bih_ref[...]
    gh = jnp.dot(h, whh_ref[...], preferred_element_type=jnp.float32) + bhh_ref[...]
    r = jax.nn.sigmoid(gi[:, :D] + gh[:, :D])
    z = jax.nn.sigmoid(gi[:, D:2 * D] + gh[:, D:2 * D])
    n = jnp.tanh(gi[:, 2 * D:] + r * gh[:, 2 * D:])
    return (1.0 - z) * n + z * h


def _gru_mm_body(parts_ref, h_ref, wih_ref, whh_ref, bih_ref, bhh_ref,
                 wnext_ref, h_out, m_out):
    hn = _gru_block(parts_ref, h_ref, wih_ref, whh_ref, bih_ref, bhh_ref)
    h_out[...] = hn
    m_out[...] = jnp.dot(hn, wnext_ref[...], preferred_element_type=jnp.float32)


def _gru_mm(parts, h, wihT, whhT, bih, bhh, wnext):
    return pl.pallas_call(
        _gru_mm_body,
        grid=(_NBLK,),
        in_specs=[pl.BlockSpec((_NC, _BLK, D), lambda i: (0, i, 0)),
                  pl.BlockSpec((_BLK, D), lambda i: (i, 0)),
                  pl.BlockSpec((D, 3 * D), lambda i: (0, 0)),
                  pl.BlockSpec((D, 3 * D), lambda i: (0, 0)),
                  pl.BlockSpec((1, 3 * D), lambda i: (0, 0)),
                  pl.BlockSpec((1, 3 * D), lambda i: (0, 0)),
                  pl.BlockSpec((D, D), lambda i: (0, 0))],
        out_specs=[pl.BlockSpec((_BLK, D), lambda i: (i, 0)),
                   pl.BlockSpec((_BLK, D), lambda i: (i, 0))],
        out_shape=[jax.ShapeDtypeStruct((N_NODES, D), jnp.float32),
                   jax.ShapeDtypeStruct((N_NODES, D), jnp.float32)],
    )(parts, h, wihT, whhT, bih, bhh, wnext)


def _gru_max_body(parts_ref, h_ref, wih_ref, whh_ref, bih_ref, bhh_ref, mx_out):
    hn = _gru_block(parts_ref, h_ref, wih_ref, whh_ref, bih_ref, bhh_ref)
    lr = jnp.where(hn >= 0.0, hn, 0.01 * hn)
    mx_out[...] = jnp.max(lr, axis=0, keepdims=True)[None]


def _gru_max(parts, h, wihT, whhT, bih, bhh):
    return pl.pallas_call(
        _gru_max_body,
        grid=(_NBLK,),
        in_specs=[pl.BlockSpec((_NC, _BLK, D), lambda i: (0, i, 0)),
                  pl.BlockSpec((_BLK, D), lambda i: (i, 0)),
                  pl.BlockSpec((D, 3 * D), lambda i: (0, 0)),
                  pl.BlockSpec((D, 3 * D), lambda i: (0, 0)),
                  pl.BlockSpec((1, 3 * D), lambda i: (0, 0)),
                  pl.BlockSpec((1, 3 * D), lambda i: (0, 0))],
        out_specs=pl.BlockSpec((1, 1, D), lambda i: (i, 0, 0)),
        out_shape=jax.ShapeDtypeStruct((_NBLK, 1, D), jnp.float32),
    )(parts, h, wihT, whhT, bih, bhh)


def _fin_body(mx_ref, lw_ref, lb_ref, o_ref):
    g = jnp.max(mx_ref[...], axis=0, keepdims=True)
    o_ref[...] = jax.nn.sigmoid(
        jnp.dot(g, lw_ref[...], preferred_element_type=jnp.float32) + lb_ref[...])


def _fin(mx, lwT, lb):
    return pl.pallas_call(
        _fin_body,
        out_shape=jax.ShapeDtypeStruct((1, D), jnp.float32),
    )(mx, lwT, lb)


def kernel(x, edge_index, ggc_weight, gru_w_ih, gru_w_hh, gru_b_ih, gru_b_hh,
           lin_w, lin_b):
    srcb = edge_index[0].reshape(_NW * _NB, _EB)
    dstb = edge_index[1].reshape(_NW * _NB, _EB)
    wihT = gru_w_ih.T
    whhT = gru_w_hh.T
    bih = gru_b_ih.reshape(1, 3 * D)
    bhh = gru_b_hh.reshape(1, 3 * D)

    h = x
    m = _mm(h, ggc_weight[0])
    parts = _mp(m, srcb, dstb)
    h, m = _gru_mm(parts, h, wihT, whhT, bih, bhh, ggc_weight[1])
    parts = _mp(m, srcb, dstb)
    mx = _gru_max(parts, h, wihT, whhT, bih, bhh).reshape(_NBLK, D)
    return _fin(mx, lin_w.T, lin_b.reshape(1, D))


# speedup vs baseline: 10.4294x; 1.0585x over previous
import jax
import jax.numpy as jnp
from jax import lax
from jax.experimental import pallas as pl
from jax.experimental.pallas import tpu as pltpu
from jax.experimental.pallas import tpu_sc as plsc

N_NODES = 10000
N_EDGES = 320000
D = 128

_NC = 2
_NS = 16
_NW = _NC * _NS
_EB = 80
_EPW = N_EDGES // _NW
_NB = _EPW // _EB
_RPT = N_NODES // _NS
_ZC = 25
_NZ = _RPT // _ZC

_BLK = 400
_NBLK = N_NODES // _BLK


def _mp_body(m_hbm, srcb_hbm, dstb_hbm, out_hbm,
             agg_sh, zb,
             si0, si1, si2, si3, si4, si5,
             di0, di1, di2, di3, di4, di5,
             rows0, rows1, rows2, rows3,
             isem0, isem1, isem2, isem3, isem4, isem5,
             gsem0, gsem1, gsem2, gsem3,
             ssem0, ssem1, ssem2, ssem3):
    c = lax.axis_index("c")
    s = lax.axis_index("s")
    wid = s * _NC + c
    row0 = wid * _NB

    si = (si0, si1, si2, si3, si4, si5)
    di = (di0, di1, di2, di3, di4, di5)
    isem = (isem0, isem1, isem2, isem3, isem4, isem5)
    rows = (rows0, rows1, rows2, rows3)
    gsem = (gsem0, gsem1, gsem2, gsem3)
    ssem = (ssem0, ssem1, ssem2, ssem3)

    def i_start(g, k):
        pltpu.async_copy(srcb_hbm.at[row0 + g], si[k], isem[k])
        pltpu.async_copy(dstb_hbm.at[row0 + g], di[k], isem[k])

    def i_wait(g, k):
        pltpu.make_async_copy(srcb_hbm.at[row0 + g], si[k], isem[k]).wait()
        pltpu.make_async_copy(dstb_hbm.at[row0 + g], di[k], isem[k]).wait()

    def g_start(g, b, k):
        pltpu.async_copy(m_hbm.at[si[k]], rows[b], gsem[b])

    def g_wait(g, b, k):
        pltpu.make_async_copy(m_hbm.at[si[k]], rows[b], gsem[b]).wait()

    def s_start(g, b, k):
        pltpu.async_copy(rows[b], agg_sh.at[di[k]], ssem[b], add=True)

    def s_wait(g, b, k):
        pltpu.make_async_copy(rows[b], agg_sh.at[di[k]], ssem[b]).wait()

    for g in range(4):
        i_start(g, g % 6)
    zv = jnp.zeros((16,), jnp.float32)

    def _zrow(r, carry):
        for c16 in range(D // 16):
            zb[r, pl.ds(c16 * 16, 16)] = zv
        return carry

    lax.fori_loop(0, _ZC, _zrow, 0)
    base_row = s * _RPT
    for k in range(_NZ):
        pltpu.sync_copy(zb, agg_sh.at[pl.ds(base_row + k * _ZC, _ZC)])
    plsc.subcore_barrier()

    i_wait(0, 0)
    g_start(0, 0, 0)
    i_wait(1, 1)
    g_start(1, 1, 1)
    for g in range(2):
        g_wait(g, g % 4, g % 6)
        s_start(g, g % 4, g % 6)
        i_wait(g + 2, (g + 2) % 6)
        g_start(g + 2, (g + 2) % 4, (g + 2) % 6)
        i_start(g + 4, (g + 4) % 6)

    niter = (_NB - 6) // 12
    def _body(it, carry):
        g0 = 2 + 12 * it
        for j in range(12):
            gg = g0 + j
            b = (2 + j) % 4
            k = (2 + j) % 6
            g_wait(gg, b, k)
            s_start(gg, b, k)
            s_wait(gg - 2, j % 4, j % 6)
            i_wait(gg + 2, (4 + j) % 6)
            g_start(gg + 2, j % 4, (4 + j) % 6)
            i_start(gg + 4, j % 6)
        return carry

    lax.fori_loop(0, niter, _body, 0)

    for gg in range(2 + 12 * niter, _NB):
        g_wait(gg, gg % 4, gg % 6)
        s_start(gg, gg % 4, gg % 6)
        s_wait(gg - 2, (gg - 2) % 4, (gg - 2) % 6)
        if gg + 2 < _NB:
            i_wait(gg + 2, (gg + 2) % 6)
            g_start(gg + 2, (gg + 2) % 4, (gg + 2) % 6)
        if gg + 4 < _NB:
            i_start(gg + 4, (gg + 4) % 6)
    for gg in range(_NB - 2, _NB):
        s_wait(gg, gg % 4, gg % 6)

    plsc.subcore_barrier()
    w0 = pl.multiple_of(s * 624, 8)

    @pl.when(s < _NS - 1)
    def _():
        pltpu.sync_copy(agg_sh.at[pl.ds(w0, 624)],
                        out_hbm.at[c, pl.ds(w0, 624)])

    @pl.when(s == _NS - 1)
    def _():
        pltpu.sync_copy(agg_sh.at[pl.ds((_NS - 1) * 624, N_NODES - (_NS - 1) * 624)],
                        out_hbm.at[c, pl.ds((_NS - 1) * 624, N_NODES - (_NS - 1) * 624)])


_mp = pl.kernel(
    _mp_body,
    out_type=jax.ShapeDtypeStruct((_NC, N_NODES, D), jnp.float32),
    mesh=plsc.VectorSubcoreMesh(core_axis_name="c", subcore_axis_name="s"),
    scratch_types=(
        [pltpu.VMEM_SHARED((N_NODES, D), jnp.float32)]
        + [pltpu.VMEM((_ZC, D), jnp.float32)]
        + [pltpu.VMEM((_EB,), jnp.int32)] * 12
        + [pltpu.VMEM((_EB, D), jnp.float32)] * 4
        + [pltpu.SemaphoreType.DMA] * 14
    ),
)


def _gru_block(parts_ref, h_ref, w_ref, wih_ref, whh_ref, bih_ref, bhh_ref):
    sums = parts_ref[0] + parts_ref[1]
    agg = jnp.dot(sums, w_ref[...], preferred_element_type=jnp.float32)
    h = h_ref[...]
    gi = jnp.dot(agg, wih_ref[...], preferred_element_type=jnp.float32) + bih_ref[...]
    gh = jnp.dot(h, whh_ref[...], preferred_element_type=jnp.float32) + bhh_ref[...]
    r = jax.nn.sigmoid(gi[:, :D] + gh[:, :D])
    z = jax.nn.sigmoid(gi[:, D:2 * D] + gh[:, D:2 * D])
    n = jnp.tanh(gi[:, 2 * D:] + r * gh[:, 2 * D:])
    return (1.0 - z) * n + z * h


def _gru_body(parts_ref, h_ref, w_ref, wih_ref, whh_ref, bih_ref, bhh_ref,
              h_out):
    h_out[...] = _gru_block(parts_ref, h_ref, w_ref, wih_ref, whh_ref,
                            bih_ref, bhh_ref)


_GRU_IN_SPECS = [pl.BlockSpec((_NC, _BLK, D), lambda i: (0, i, 0)),
                 pl.BlockSpec((_BLK, D), lambda i: (i, 0)),
                 pl.BlockSpec((D, D), lambda i: (0, 0)),
                 pl.BlockSpec((D, 3 * D), lambda i: (0, 0)),
                 pl.BlockSpec((D, 3 * D), lambda i: (0, 0)),
                 pl.BlockSpec((1, 3 * D), lambda i: (0, 0)),
                 pl.BlockSpec((1, 3 * D), lambda i: (0, 0))]


def _gru(parts, h, w, wihT, whhT, bih, bhh):
    return pl.pallas_call(
        _gru_body,
        grid=(_NBLK,),
        in_specs=_GRU_IN_SPECS,
        out_specs=pl.BlockSpec((_BLK, D), lambda i: (i, 0)),
        out_shape=jax.ShapeDtypeStruct((N_NODES, D), jnp.float32),
    )(parts, h, w, wihT, whhT, bih, bhh)


def _gru_fin_body(parts_ref, h_ref, w_ref, wih_ref, whh_ref, bih_ref, bhh_ref,
                  lw_ref, lb_ref, o_ref, mx_sc):
    i = pl.program_id(0)
    hn = _gru_block(parts_ref, h_ref, w_ref, wih_ref, whh_ref, bih_ref, bhh_ref)
    lr = jnp.where(hn >= 0.0, hn, 0.01 * hn)
    bmax = jnp.max(lr, axis=0, keepdims=True)

    @pl.when(i == 0)
    def _():
        mx_sc[...] = bmax

    @pl.when(i > 0)
    def _():
        mx_sc[...] = jnp.maximum(mx_sc[...], bmax)

    @pl.when(i == _NBLK - 1)
    def _():
        o_ref[...] = jax.nn.sigmoid(
            jnp.dot(mx_sc[...], lw_ref[...],
                    preferred_element_type=jnp.float32) + lb_ref[...])


def _gru_fin(parts, h, w, wihT, whhT, bih, bhh, lwT, lb):
    return pl.pallas_call(
        _gru_fin_body,
        grid=(_NBLK,),
        in_specs=_GRU_IN_SPECS + [pl.BlockSpec((D, D), lambda i: (0, 0)),
                                  pl.BlockSpec((1, D), lambda i: (0, 0))],
        out_specs=pl.BlockSpec((1, D), lambda i: (0, 0)),
        out_shape=jax.ShapeDtypeStruct((1, D), jnp.float32),
        scratch_shapes=[pltpu.VMEM((1, D), jnp.float32)],
    )(parts, h, w, wihT, whhT, bih, bhh, lwT, lb)


def kernel(x, edge_index, ggc_weight, gru_w_ih, gru_w_hh, gru_b_ih, gru_b_hh,
           lin_w, lin_b):
    srcb = edge_index[0].reshape(_NW * _NB, _EB)
    dstb = edge_index[1].reshape(_NW * _NB, _EB)
    wihT = gru_w_ih.T
    whhT = gru_w_hh.T
    bih = gru_b_ih.reshape(1, 3 * D)
    bhh = gru_b_hh.reshape(1, 3 * D)

    parts = _mp(x, srcb, dstb)
    h = _gru(parts, x, ggc_weight[0], wihT, whhT, bih, bhh)
    parts = _mp(h, srcb, dstb)
    return _gru_fin(parts, h, ggc_weight[1], wihT, whhT, bih, bhh,
                    lin_w.T, lin_b.reshape(1, D))


# speedup vs baseline: 11.0121x; 1.0559x over previous
import jax
import jax.numpy as jnp
from jax import lax
from jax.experimental import pallas as pl
from jax.experimental.pallas import tpu as pltpu
from jax.experimental.pallas import tpu_sc as plsc

N_NODES = 10000
N_EDGES = 320000
D = 128

_NC = 2
_NS = 16
_NW = _NC * _NS
_EB = 80
_EPW = N_EDGES // _NW
_NB = _EPW // _EB
_RPT = N_NODES // _NS
_ZC = 25
_NZ = _RPT // _ZC

_BLK = 400
_NBLK = N_NODES // _BLK


def _mp_body(m_hbm, eflat_hbm, out_hbm,
             agg_sh, zb,
             si0, si1, si2, si3, si4, si5,
             di0, di1, di2, di3, di4, di5,
             rows0, rows1, rows2, rows3,
             isem0, isem1, isem2, isem3, isem4, isem5,
             gsem0, gsem1, gsem2, gsem3,
             ssem0, ssem1, ssem2, ssem3):
    c = lax.axis_index("c")
    s = lax.axis_index("s")
    wid = s * _NC + c
    row0 = wid * _NB

    si = (si0, si1, si2, si3, si4, si5)
    di = (di0, di1, di2, di3, di4, di5)
    isem = (isem0, isem1, isem2, isem3, isem4, isem5)
    rows = (rows0, rows1, rows2, rows3)
    gsem = (gsem0, gsem1, gsem2, gsem3)
    ssem = (ssem0, ssem1, ssem2, ssem3)

    def i_start(g, k):
        off = pl.multiple_of((row0 + g) * _EB, 8)
        pltpu.async_copy(eflat_hbm.at[pl.ds(off, _EB)], si[k], isem[k])
        pltpu.async_copy(eflat_hbm.at[pl.ds(N_EDGES + off, _EB)], di[k], isem[k])

    def i_wait(g, k):
        off = pl.multiple_of((row0 + g) * _EB, 8)
        pltpu.make_async_copy(eflat_hbm.at[pl.ds(off, _EB)], si[k], isem[k]).wait()
        pltpu.make_async_copy(eflat_hbm.at[pl.ds(N_EDGES + off, _EB)], di[k],
                              isem[k]).wait()

    def g_start(g, b, k):
        pltpu.async_copy(m_hbm.at[si[k]], rows[b], gsem[b])

    def g_wait(g, b, k):
        pltpu.make_async_copy(m_hbm.at[si[k]], rows[b], gsem[b]).wait()

    def s_start(g, b, k):
        pltpu.async_copy(rows[b], agg_sh.at[di[k]], ssem[b], add=True)

    def s_wait(g, b, k):
        pltpu.make_async_copy(rows[b], agg_sh.at[di[k]], ssem[b]).wait()

    for g in range(4):
        i_start(g, g % 6)
    zv = jnp.zeros((16,), jnp.float32)

    def _zrow(r, carry):
        for c16 in range(D // 16):
            zb[r, pl.ds(c16 * 16, 16)] = zv
        return carry

    lax.fori_loop(0, _ZC, _zrow, 0)
    i_wait(0, 0)
    g_start(0, 0, 0)
    i_wait(1, 1)
    g_start(1, 1, 1)
    base_row = s * _RPT
    for k in range(_NZ):
        pltpu.async_copy(zb, agg_sh.at[pl.ds(base_row + k * _ZC, _ZC)], isem[5])
    for k in range(_NZ):
        pltpu.make_async_copy(zb, agg_sh.at[pl.ds(base_row + k * _ZC, _ZC)],
                              isem[5]).wait()
    plsc.subcore_barrier()

    for g in range(2):
        g_wait(g, g % 4, g % 6)
        s_start(g, g % 4, g % 6)
        i_wait(g + 2, (g + 2) % 6)
        g_start(g + 2, (g + 2) % 4, (g + 2) % 6)
        i_start(g + 4, (g + 4) % 6)

    niter = (_NB - 6) // 12
    def _body(it, carry):
        g0 = 2 + 12 * it
        for j in range(12):
            gg = g0 + j
            b = (2 + j) % 4
            k = (2 + j) % 6
            g_wait(gg, b, k)
            s_start(gg, b, k)
            s_wait(gg - 2, j % 4, j % 6)
            i_wait(gg + 2, (4 + j) % 6)
            g_start(gg + 2, j % 4, (4 + j) % 6)
            i_start(gg + 4, j % 6)
        return carry

    lax.fori_loop(0, niter, _body, 0)

    for gg in range(2 + 12 * niter, _NB):
        g_wait(gg, gg % 4, gg % 6)
        s_start(gg, gg % 4, gg % 6)
        s_wait(gg - 2, (gg - 2) % 4, (gg - 2) % 6)
        if gg + 2 < _NB:
            i_wait(gg + 2, (gg + 2) % 6)
            g_start(gg + 2, (gg + 2) % 4, (gg + 2) % 6)
        if gg + 4 < _NB:
            i_start(gg + 4, (gg + 4) % 6)
    for gg in range(_NB - 2, _NB):
        s_wait(gg, gg % 4, gg % 6)

    plsc.subcore_barrier()
    w0 = pl.multiple_of(s * 624, 8)

    @pl.when(s < _NS - 1)
    def _():
        pltpu.sync_copy(agg_sh.at[pl.ds(w0, 624)],
                        out_hbm.at[c, pl.ds(w0, 624)])

    @pl.when(s == _NS - 1)
    def _():
        pltpu.sync_copy(agg_sh.at[pl.ds((_NS - 1) * 624, N_NODES - (_NS - 1) * 624)],
                        out_hbm.at[c, pl.ds((_NS - 1) * 624, N_NODES - (_NS - 1) * 624)])


_mp = pl.kernel(
    _mp_body,
    out_type=jax.ShapeDtypeStruct((_NC, N_NODES, D), jnp.float32),
    mesh=plsc.VectorSubcoreMesh(core_axis_name="c", subcore_axis_name="s"),
    scratch_types=(
        [pltpu.VMEM_SHARED((N_NODES, D), jnp.float32)]
        + [pltpu.VMEM((_ZC, D), jnp.float32)]
        + [pltpu.VMEM((_EB,), jnp.int32)] * 12
        + [pltpu.VMEM((_EB, D), jnp.float32)] * 4
        + [pltpu.SemaphoreType.DMA] * 14
    ),
)


def _dot_t(a, b):
    return lax.dot_general(a, b, (((1,), (1,)), ((), ())),
                           preferred_element_type=jnp.float32)


def _gru_block(parts_ref, h_ref, w_ref, wih_ref, whh_ref, bih_ref, bhh_ref):
    sums = parts_ref[0] + parts_ref[1]
    agg = jnp.dot(sums, w_ref[...], preferred_element_type=jnp.float32)
    h = h_ref[...]
    gi = _dot_t(agg, wih_ref[...]) + bih_ref[...]
    gh = _dot_t(h, whh_ref[...]) + bhh_ref[...]
    r = jax.nn.sigmoid(gi[:, :D] + gh[:, :D])
    z = jax.nn.sigmoid(gi[:, D:2 * D] + gh[:, D:2 * D])
    n = jnp.tanh(gi[:, 2 * D:] + r * gh[:, 2 * D:])
    return (1.0 - z) * n + z * h


def _gru_body(parts_ref, h_ref, w_ref, wih_ref, whh_ref, bih_ref, bhh_ref,
              h_out):
    h_out[...] = _gru_block(parts_ref, h_ref, w_ref, wih_ref, whh_ref,
                            bih_ref, bhh_ref)


_GRU_IN_SPECS = [pl.BlockSpec((_NC, _BLK, D), lambda i: (0, i, 0)),
                 pl.BlockSpec((_BLK, D), lambda i: (i, 0)),
                 pl.BlockSpec((D, D), lambda i: (0, 0)),
                 pl.BlockSpec((3 * D, D), lambda i: (0, 0)),
                 pl.BlockSpec((3 * D, D), lambda i: (0, 0)),
                 pl.BlockSpec((1, 3 * D), lambda i: (0, 0)),
                 pl.BlockSpec((1, 3 * D), lambda i: (0, 0))]


def _gru(parts, h, w, wihT, whhT, bih, bhh):
    return pl.pallas_call(
        _gru_body,
        grid=(_NBLK,),
        in_specs=_GRU_IN_SPECS,
        out_specs=pl.BlockSpec((_BLK, D), lambda i: (i, 0)),
        out_shape=jax.ShapeDtypeStruct((N_NODES, D), jnp.float32),
    )(parts, h, w, wihT, whhT, bih, bhh)


def _gru_fin_body(parts_ref, h_ref, w_ref, wih_ref, whh_ref, bih_ref, bhh_ref,
                  lw_ref, lb_ref, o_ref, mx_sc):
    i = pl.program_id(0)
    hn = _gru_block(parts_ref, h_ref, w_ref, wih_ref, whh_ref, bih_ref, bhh_ref)
    lr = jnp.where(hn >= 0.0, hn, 0.01 * hn)
    bmax = jnp.max(lr, axis=0, keepdims=True)

    @pl.when(i == 0)
    def _():
        mx_sc[...] = bmax

    @pl.when(i > 0)
    def _():
        mx_sc[...] = jnp.maximum(mx_sc[...], bmax)

    @pl.when(i == _NBLK - 1)
    def _():
        o_ref[...] = jax.nn.sigmoid(_dot_t(mx_sc[...], lw_ref[...]) + lb_ref[...])


def _gru_fin(parts, h, w, wihT, whhT, bih, bhh, lwT, lb):
    return pl.pallas_call(
        _gru_fin_body,
        grid=(_NBLK,),
        in_specs=_GRU_IN_SPECS + [pl.BlockSpec((D, D), lambda i: (0, 0)),
                                  pl.BlockSpec((1, D), lambda i: (0, 0))],
        out_specs=pl.BlockSpec((1, D), lambda i: (0, 0)),
        out_shape=jax.ShapeDtypeStruct((1, D), jnp.float32),
        scratch_shapes=[pltpu.VMEM((1, D), jnp.float32)],
    )(parts, h, w, wihT, whhT, bih, bhh, lwT, lb)


def kernel(x, edge_index, ggc_weight, gru_w_ih, gru_w_hh, gru_b_ih, gru_b_hh,
           lin_w, lin_b):
    eflat = edge_index.reshape(2 * N_EDGES)
    bih = gru_b_ih.reshape(1, 3 * D)
    bhh = gru_b_hh.reshape(1, 3 * D)

    parts = _mp(x, eflat)
    h = _gru(parts, x, ggc_weight[0], gru_w_ih, gru_w_hh, bih, bhh)
    parts = _mp(h, eflat)
    return _gru_fin(parts, h, ggc_weight[1], gru_w_ih, gru_w_hh, bih, bhh,
                    lin_w, lin_b.reshape(1, D))
